# Initial kernel scaffold; baseline (speedup 1.0000x reference)
import functools

import jax
import jax.numpy as jnp
from jax import lax
from jax.experimental import pallas as pl
from jax.experimental.pallas import tpu as pltpu
from jax.experimental.pallas import tpu_sc as plsc

_N_SEG = 10000
_NNZ = 320000
_D = 128
_EPS = 1e-16

_NC = 2
_NS = 16
_NW = _NC * _NS
_CHUNK = _NNZ // _NW
_BLK = 80
_NBLK = _CHUNK // _BLK
_SEG_SLICE = _N_SEG // _NS
_CW = 16


def _sc_scatter(values, seg):
    mesh = plsc.VectorSubcoreMesh(core_axis_name="c", subcore_axis_name="s")

    @functools.partial(
        pl.kernel,
        out_type=(
            jax.ShapeDtypeStruct((_NC, _N_SEG, _D), jnp.float32),
            jax.ShapeDtypeStruct((_NC, _N_SEG, _CW), jnp.float32),
        ),
        mesh=mesh,
        scratch_types=[
            pltpu.VMEM((_BLK, _D), jnp.float32),
            pltpu.VMEM((_BLK,), jnp.int32),
            pltpu.VMEM((_BLK, _CW), jnp.float32),
            pltpu.VMEM((_SEG_SLICE, _D), jnp.float32),
            pltpu.VMEM((_SEG_SLICE, _CW), jnp.float32),
            pltpu.VMEM_SHARED((_N_SEG, _D), jnp.float32),
            pltpu.VMEM_SHARED((_N_SEG, _CW), jnp.float32),
        ],
    )
    def k(values_hbm, seg_hbm, pooled_out, counts_out,
          val_b, idx_b, ones_b, zrow_b, zcnt_b, pooled_sh, counts_sh):
        c = lax.axis_index("c")
        s = lax.axis_index("s")
        base = (c * _NS + s) * _CHUNK

        zeros16 = jnp.zeros((16,), jnp.float32)
        ones16 = jnp.ones((16,), jnp.float32)

        def fill_zero(r, _):
            for j in range(_D // 16):
                zrow_b[r, pl.ds(j * 16, 16)] = zeros16
            zcnt_b[r, pl.ds(0, 16)] = zeros16
            return _

        lax.fori_loop(0, _SEG_SLICE, fill_zero, None)

        def fill_ones(r, _):
            ones_b[r, pl.ds(0, 16)] = ones16
            return _

        lax.fori_loop(0, _BLK, fill_ones, None)

        pltpu.sync_copy(zrow_b, pooled_sh.at[pl.ds(s * _SEG_SLICE, _SEG_SLICE)])
        pltpu.sync_copy(zcnt_b, counts_sh.at[pl.ds(s * _SEG_SLICE, _SEG_SLICE)])
        plsc.subcore_barrier()

        def body(i, _):
            off = base + i * _BLK
            pltpu.sync_copy(seg_hbm.at[pl.ds(off, _BLK)], idx_b)
            pltpu.sync_copy(values_hbm.at[pl.ds(off, _BLK)], val_b)
            pltpu.sync_copy(val_b, pooled_sh.at[idx_b], add=True)
            pltpu.sync_copy(ones_b, counts_sh.at[idx_b], add=True)
            return _

        lax.fori_loop(0, _NBLK, body, None)
        plsc.subcore_barrier()

        row0 = s * _SEG_SLICE
        pltpu.sync_copy(pooled_sh.at[pl.ds(row0, _SEG_SLICE)],
                        pooled_out.at[c, pl.ds(row0, _SEG_SLICE)])
        pltpu.sync_copy(counts_sh.at[pl.ds(row0, _SEG_SLICE)],
                        counts_out.at[c, pl.ds(row0, _SEG_SLICE)])

    return k(values, seg)


def _tc_merge(p0, p1, c0, c1):
    blk = 1250

    def k(p0_ref, p1_ref, c0_ref, c1_ref, o_ref):
        n = c0_ref[...] + c1_ref[...] + _EPS
        o_ref[...] = (p0_ref[...] + p1_ref[...]) / n

    grid = _N_SEG // blk
    return pl.pallas_call(
        k,
        grid=(grid,),
        in_specs=[
            pl.BlockSpec((blk, _D), lambda i: (i, 0)),
            pl.BlockSpec((blk, _D), lambda i: (i, 0)),
            pl.BlockSpec((blk, 1), lambda i: (i, 0)),
            pl.BlockSpec((blk, 1), lambda i: (i, 0)),
        ],
        out_specs=pl.BlockSpec((blk, _D), lambda i: (i, 0)),
        out_shape=jax.ShapeDtypeStruct((_N_SEG, _D), jnp.float32),
    )(p0, p1, c0, c1)


@jax.jit
def kernel(tens_values, tens_indices):
    seg = tens_indices[1].astype(jnp.int32)
    pooled, counts = _sc_scatter(tens_values, seg)
    return _tc_merge(pooled[0], pooled[1],
                     counts[0, :, 0:1], counts[1, :, 0:1])


# speedup vs baseline: 4.5781x; 4.5781x over previous
import functools

import jax
import jax.numpy as jnp
from jax import lax
from jax.experimental import pallas as pl
from jax.experimental.pallas import tpu as pltpu
from jax.experimental.pallas import tpu_sc as plsc

_N_SEG = 10000
_NNZ = 320000
_D = 128
_EPS = 1e-16

_NC = 2
_NS = 16
_NW = _NC * _NS
_CHUNK = _NNZ // _NW
_BLK = 80
_NBLK = _CHUNK // _BLK
_SEGB = _N_SEG // _BLK


def _sc_scatter(values, seg):
    mesh = plsc.VectorSubcoreMesh(core_axis_name="c", subcore_axis_name="s")

    @functools.partial(
        pl.kernel,
        out_type=(
            jax.ShapeDtypeStruct((_NC, _SEGB, _BLK, _D), jnp.float32),
            jax.ShapeDtypeStruct((_NC, _NS, _N_SEG), jnp.float32),
        ),
        mesh=mesh,
        scratch_types=[
            pltpu.VMEM((_BLK, _D), jnp.float32),
            pltpu.VMEM((_BLK,), jnp.int32),
            pltpu.VMEM((_N_SEG,), jnp.float32),
            pltpu.VMEM_SHARED((_N_SEG, _D), jnp.float32),
        ],
        compiler_params=pltpu.CompilerParams(needs_layout_passes=False),
    )
    def k(values_hbm, seg_hbm, pooled_out, counts_out,
          val_b, idx_b, hist_b, pooled_sh):
        c = lax.axis_index("c")
        s = lax.axis_index("s")
        base = (c * _NS + s) * _CHUNK

        zeros16 = jnp.zeros((16,), jnp.float32)
        ones16 = jnp.ones((16,), jnp.float32)

        def fill_zero(r, _):
            for j in range(_D // 16):
                val_b[r, pl.ds(j * 16, 16)] = zeros16
            return _

        lax.fori_loop(0, _BLK, fill_zero, None)

        def fill_hist(r, _):
            hist_b[pl.ds(r * 16, 16)] = zeros16
            return _

        lax.fori_loop(0, _N_SEG // 16, fill_hist, None)

        def zero_blocks(t, _):
            blk = t * _NS + s

            @pl.when(blk < _SEGB)
            def _do():
                pltpu.sync_copy(val_b, pooled_sh.at[pl.ds(blk * _BLK, _BLK)])

            return _

        lax.fori_loop(0, (_SEGB + _NS - 1) // _NS, zero_blocks, None)
        plsc.subcore_barrier()

        def body(i, _):
            off = base + i * _BLK
            pltpu.sync_copy(seg_hbm.at[pl.ds(off, _BLK)], idx_b)
            pltpu.sync_copy(values_hbm.at[pl.ds(off, _BLK)], val_b)
            pltpu.sync_copy(val_b, pooled_sh.at[idx_b], add=True)
            for j in range(_BLK // 16):
                idxv = idx_b[pl.ds(j * 16, 16)]
                plsc.addupdate_scatter(hist_b, [idxv], ones16)
            return _

        lax.fori_loop(0, _NBLK, body, None)
        plsc.subcore_barrier()

        def writeback(t, _):
            blk = t * _NS + s

            @pl.when(blk < _SEGB)
            def _do():
                pltpu.sync_copy(pooled_sh.at[pl.ds(blk * _BLK, _BLK)], val_b)
                pltpu.sync_copy(val_b, pooled_out.at[c, blk])

            return _

        lax.fori_loop(0, (_SEGB + _NS - 1) // _NS, writeback, None)
        pltpu.sync_copy(hist_b, counts_out.at[c, s])

    return k(values, seg)


def _tc_merge(p0, p1, cnt_t):
    blk = 1000

    def k(p0_ref, p1_ref, c_ref, o_ref):
        n = jnp.sum(c_ref[...], axis=1, keepdims=True) + _EPS
        o_ref[...] = (p0_ref[...] + p1_ref[...]) / n

    grid = _N_SEG // blk
    return pl.pallas_call(
        k,
        grid=(grid,),
        in_specs=[
            pl.BlockSpec((blk, _D), lambda i: (i, 0)),
            pl.BlockSpec((blk, _D), lambda i: (i, 0)),
            pl.BlockSpec((blk, _NW), lambda i: (i, 0)),
        ],
        out_specs=pl.BlockSpec((blk, _D), lambda i: (i, 0)),
        out_shape=jax.ShapeDtypeStruct((_N_SEG, _D), jnp.float32),
    )(p0, p1, cnt_t)


@jax.jit
def kernel(tens_values, tens_indices):
    seg = tens_indices[1].astype(jnp.int32)
    pooled, counts = _sc_scatter(tens_values, seg)
    pooled = pooled.reshape(_NC, _N_SEG, _D)
    cnt_t = counts.reshape(_NW, _N_SEG).transpose(1, 0)
    return _tc_merge(pooled[0], pooled[1], cnt_t)


# speedup vs baseline: 6.9961x; 1.5282x over previous
import functools

import jax
import jax.numpy as jnp
from jax import lax
from jax.experimental import pallas as pl
from jax.experimental.pallas import tpu as pltpu
from jax.experimental.pallas import tpu_sc as plsc

_N_SEG = 10000
_NNZ = 320000
_D = 128
_EPS = 1e-16

_NC = 2
_NS = 16
_NW = _NC * _NS
_CHUNK = _NNZ // _NW
_BLK = 80
_NBLK = _CHUNK // _BLK
_SEGB = _N_SEG // _BLK


def _sc_scatter(values, seg):
    mesh = plsc.VectorSubcoreMesh(core_axis_name="c", subcore_axis_name="s")

    @functools.partial(
        pl.kernel,
        out_type=(
            jax.ShapeDtypeStruct((_NC, _SEGB, _BLK, _D), jnp.float32),
            jax.ShapeDtypeStruct((_NC, _NS, _N_SEG), jnp.float32),
        ),
        mesh=mesh,
        scratch_types=[
            pltpu.VMEM((_BLK, _D), jnp.float32),
            pltpu.VMEM((_BLK, _D), jnp.float32),
            pltpu.VMEM((_BLK,), jnp.int32),
            pltpu.VMEM((_BLK,), jnp.int32),
            pltpu.VMEM((_N_SEG,), jnp.float32),
            pltpu.VMEM_SHARED((_N_SEG, _D), jnp.float32),
            pltpu.SemaphoreType.DMA,
            pltpu.SemaphoreType.DMA,
        ],
        compiler_params=pltpu.CompilerParams(needs_layout_passes=False),
    )
    def k(values_hbm, seg_hbm, pooled_out, counts_out,
          val_b0, val_b1, idx_b0, idx_b1, hist_b, pooled_sh, lsem0, lsem1):
        c = lax.axis_index("c")
        s = lax.axis_index("s")
        base = (c * _NS + s) * _CHUNK

        val_b = (val_b0, val_b1)
        idx_b = (idx_b0, idx_b1)
        lsem = (lsem0, lsem1)

        zeros16 = jnp.zeros((16,), jnp.float32)
        ones16 = jnp.ones((16,), jnp.float32)

        def start_load(i, b):
            off = base + i * _BLK
            pltpu.async_copy(seg_hbm.at[pl.ds(off, _BLK)], idx_b[b], lsem[b])
            pltpu.async_copy(values_hbm.at[pl.ds(off, _BLK)], val_b[b], lsem[b])

        def wait_load(b):
            pltpu.make_async_copy(seg_hbm.at[pl.ds(0, _BLK)],
                                  idx_b[b], lsem[b]).wait()
            pltpu.make_async_copy(values_hbm.at[pl.ds(0, _BLK)],
                                  val_b[b], lsem[b]).wait()

        start_load(0, 0)

        def fill_zero(r, _):
            for j in range(_D // 16):
                val_b1[r, pl.ds(j * 16, 16)] = zeros16
            return _

        lax.fori_loop(0, _BLK, fill_zero, None)

        def fill_hist(r, _):
            hist_b[pl.ds(r * 16, 16)] = zeros16
            return _

        lax.fori_loop(0, _N_SEG // 16, fill_hist, None)

        def zero_blocks(t, _):
            blk = t * _NS + s

            @pl.when(blk < _SEGB)
            def _do():
                pltpu.sync_copy(val_b1, pooled_sh.at[pl.ds(blk * _BLK, _BLK)])

            return _

        lax.fori_loop(0, (_SEGB + _NS - 1) // _NS, zero_blocks, None)
        plsc.subcore_barrier()

        def process(i, b, prefetch):
            wait_load(b)
            if prefetch:
                start_load(i + 1, 1 - b)
            pltpu.sync_copy(val_b[b], pooled_sh.at[idx_b[b]], add=True)
            for j in range(_BLK // 16):
                idxv = idx_b[b][pl.ds(j * 16, 16)]
                plsc.addupdate_scatter(hist_b, [idxv], ones16)

        def body(g, _):
            i = g * 2
            process(i, 0, True)
            process(i + 1, 1, True)
            return _

        lax.fori_loop(0, (_NBLK - 1) // 2, body, None)
        process(_NBLK - 1, 0, False)
        plsc.subcore_barrier()

        def writeback(t, _):
            blk = t * _NS + s

            @pl.when(blk < _SEGB)
            def _do():
                pltpu.sync_copy(pooled_sh.at[pl.ds(blk * _BLK, _BLK)], val_b0)
                pltpu.sync_copy(val_b0, pooled_out.at[c, blk])

            return _

        lax.fori_loop(0, (_SEGB + _NS - 1) // _NS, writeback, None)
        pltpu.sync_copy(hist_b, counts_out.at[c, s])

    return k(values, seg)


def _tc_merge(p0, p1, cnt_t):
    blk = 1000

    def k(p0_ref, p1_ref, c_ref, o_ref):
        n = jnp.sum(c_ref[...], axis=1, keepdims=True) + _EPS
        o_ref[...] = (p0_ref[...] + p1_ref[...]) / n

    grid = _N_SEG // blk
    return pl.pallas_call(
        k,
        grid=(grid,),
        in_specs=[
            pl.BlockSpec((blk, _D), lambda i: (i, 0)),
            pl.BlockSpec((blk, _D), lambda i: (i, 0)),
            pl.BlockSpec((blk, _NW), lambda i: (i, 0)),
        ],
        out_specs=pl.BlockSpec((blk, _D), lambda i: (i, 0)),
        out_shape=jax.ShapeDtypeStruct((_N_SEG, _D), jnp.float32),
    )(p0, p1, cnt_t)


@jax.jit
def kernel(tens_values, tens_indices):
    seg = tens_indices[1].astype(jnp.int32)
    pooled, counts = _sc_scatter(tens_values, seg)
    pooled = pooled.reshape(_NC, _N_SEG, _D)
    cnt_t = counts.reshape(_NW, _N_SEG).transpose(1, 0)
    return _tc_merge(pooled[0], pooled[1], cnt_t)


# speedup vs baseline: 7.3922x; 1.0566x over previous
import functools

import jax
import jax.numpy as jnp
from jax import lax
from jax.experimental import pallas as pl
from jax.experimental.pallas import tpu as pltpu
from jax.experimental.pallas import tpu_sc as plsc

_N_SEG = 10000
_NNZ = 320000
_D = 128
_EPS = 1e-16

_NC = 2
_NS = 16
_NW = _NC * _NS
_CHUNK = _NNZ // _NW
_BLK = 80
_NBLK = _CHUNK // _BLK
_SEGB = _N_SEG // _BLK
_CROWS = 80


def _sc_scatter(values, seg):
    mesh = plsc.VectorSubcoreMesh(core_axis_name="c", subcore_axis_name="s")

    @functools.partial(
        pl.kernel,
        out_type=(
            jax.ShapeDtypeStruct((_NC, _SEGB, _BLK, _D), jnp.float32),
            jax.ShapeDtypeStruct((_NC, _CROWS, _D), jnp.float32),
        ),
        mesh=mesh,
        scratch_types=[
            pltpu.VMEM((_BLK, _D), jnp.float32),
            pltpu.VMEM((_BLK, _D), jnp.float32),
            pltpu.VMEM((_BLK,), jnp.int32),
            pltpu.VMEM((_BLK,), jnp.int32),
            pltpu.VMEM((_CROWS, _D), jnp.float32),
            pltpu.VMEM((_CROWS,), jnp.int32),
            pltpu.VMEM_SHARED((_N_SEG, _D), jnp.float32),
            pltpu.VMEM_SHARED((_CROWS, _D), jnp.float32),
            pltpu.SemaphoreType.DMA,
            pltpu.SemaphoreType.DMA,
        ],
        compiler_params=pltpu.CompilerParams(needs_layout_passes=False),
    )
    def k(values_hbm, seg_hbm, pooled_out, counts_out,
          val_b0, val_b1, idx_b0, idx_b1, hist_b, lin_b,
          pooled_sh, counts_sh, lsem0, lsem1):
        c = lax.axis_index("c")
        s = lax.axis_index("s")
        base = (c * _NS + s) * _CHUNK

        val_b = (val_b0, val_b1)
        idx_b = (idx_b0, idx_b1)
        lsem = (lsem0, lsem1)

        zeros16 = jnp.zeros((16,), jnp.float32)
        ones16 = jnp.ones((16,), jnp.float32)
        iota16 = lax.iota(jnp.int32, 16)

        def start_load(i, b):
            off = base + i * _BLK
            pltpu.async_copy(seg_hbm.at[pl.ds(off, _BLK)], idx_b[b], lsem[b])
            pltpu.async_copy(values_hbm.at[pl.ds(off, _BLK)], val_b[b], lsem[b])

        def wait_load(b):
            pltpu.make_async_copy(seg_hbm.at[pl.ds(0, _BLK)],
                                  idx_b[b], lsem[b]).wait()
            pltpu.make_async_copy(values_hbm.at[pl.ds(0, _BLK)],
                                  val_b[b], lsem[b]).wait()

        start_load(0, 0)

        def fill_zero(r, _):
            for j in range(_D // 16):
                val_b1[r, pl.ds(j * 16, 16)] = zeros16
                hist_b[r, pl.ds(j * 16, 16)] = zeros16
            return _

        lax.fori_loop(0, _BLK, fill_zero, None)

        for t in range(_CROWS // 16):
            lin_b[pl.ds(t * 16, 16)] = iota16 + (16 * t)

        def zero_blocks(t, _):
            blk = t * _NS + s

            @pl.when(blk < _SEGB)
            def _do():
                pltpu.sync_copy(val_b1, pooled_sh.at[pl.ds(blk * _BLK, _BLK)])

            return _

        lax.fori_loop(0, (_SEGB + _NS - 1) // _NS, zero_blocks, None)

        @pl.when(s == 0)
        def _zero_counts():
            pltpu.sync_copy(val_b1, counts_sh)

        plsc.subcore_barrier()

        def process(i, b, prefetch):
            wait_load(b)
            if prefetch:
                start_load(i + 1, 1 - b)
            pltpu.sync_copy(val_b[b], pooled_sh.at[idx_b[b]], add=True)
            for j in range(_BLK // 16):
                idxv = idx_b[b][pl.ds(j * 16, 16)]
                row = lax.shift_right_logical(idxv, 7)
                col = lax.bitwise_and(idxv, 127)
                plsc.addupdate_scatter(hist_b, [row, col], ones16)

        def body(g, _):
            i = g * 2
            process(i, 0, True)
            process(i + 1, 1, True)
            return _

        lax.fori_loop(0, (_NBLK - 1) // 2, body, None)
        process(_NBLK - 1, 0, False)
        plsc.subcore_barrier()

        pltpu.sync_copy(hist_b, counts_sh.at[lin_b], add=True)

        def writeback(t, _):
            blk = t * _NS + s

            @pl.when(blk < _SEGB)
            def _do():
                pltpu.sync_copy(pooled_sh.at[pl.ds(blk * _BLK, _BLK)], val_b0)
                pltpu.sync_copy(val_b0, pooled_out.at[c, blk])

            return _

        lax.fori_loop(0, (_SEGB + _NS - 1) // _NS, writeback, None)
        plsc.subcore_barrier()

        @pl.when(s == 0)
        def _write_counts():
            pltpu.sync_copy(counts_sh, hist_b)
            pltpu.sync_copy(hist_b, counts_out.at[c])

    return k(values, seg)


def _tc_merge(pooled, cnt_t):
    blk = 1000

    def k(p0_ref, p1_ref, c_ref, o_ref):
        n = jnp.sum(c_ref[...], axis=1, keepdims=True) + _EPS
        o_ref[...] = (p0_ref[0] + p1_ref[0]) / n

    grid = _N_SEG // blk
    return pl.pallas_call(
        k,
        grid=(grid,),
        in_specs=[
            pl.BlockSpec((1, blk, _D), lambda i: (0, i, 0)),
            pl.BlockSpec((1, blk, _D), lambda i: (1, i, 0)),
            pl.BlockSpec((blk, _NC), lambda i: (i, 0)),
        ],
        out_specs=pl.BlockSpec((blk, _D), lambda i: (i, 0)),
        out_shape=jax.ShapeDtypeStruct((_N_SEG, _D), jnp.float32),
    )(pooled, pooled, cnt_t)


@jax.jit
def kernel(tens_values, tens_indices):
    seg = tens_indices[1].astype(jnp.int32)
    pooled, counts = _sc_scatter(tens_values, seg)
    pooled = pooled.reshape(_NC, _N_SEG, _D)
    cnt_t = counts.reshape(_NC, _CROWS * _D).transpose(1, 0)
    return _tc_merge(pooled, cnt_t)


# speedup vs baseline: 11.0440x; 1.4940x over previous
import functools

import jax
import jax.numpy as jnp
from jax import lax
from jax.experimental import pallas as pl
from jax.experimental.pallas import tpu as pltpu
from jax.experimental.pallas import tpu_sc as plsc

_N_SEG = 10000
_NNZ = 320000
_D = 128
_EPS = 1e-16

_NC = 2
_NS = 16
_NW = _NC * _NS
_CHUNK = _NNZ // _NW
_BLK = 80
_NBLK = _CHUNK // _BLK
_SEGB = _N_SEG // _BLK
_CROWS = 80
_SEG_OFF = _NNZ


def _sc_scatter(values, seg):
    mesh = plsc.VectorSubcoreMesh(core_axis_name="c", subcore_axis_name="s")

    @functools.partial(
        pl.kernel,
        out_type=(
            jax.ShapeDtypeStruct((_NC, _SEGB, _BLK, _D), jnp.float32),
            jax.ShapeDtypeStruct((_NC, _CROWS, _D), jnp.float32),
        ),
        mesh=mesh,
        scratch_types=[
            pltpu.VMEM((_BLK, _D), jnp.float32),
            pltpu.VMEM((_BLK, _D), jnp.float32),
            pltpu.VMEM((_BLK, _D), jnp.float32),
            pltpu.VMEM((_BLK,), jnp.int32),
            pltpu.VMEM((_BLK,), jnp.int32),
            pltpu.VMEM((_BLK,), jnp.int32),
            pltpu.VMEM((_CROWS, _D), jnp.float32),
            pltpu.VMEM((_CROWS,), jnp.int32),
            pltpu.VMEM_SHARED((_N_SEG, _D), jnp.float32),
            pltpu.VMEM_SHARED((_CROWS, _D), jnp.float32),
            pltpu.SemaphoreType.DMA,
            pltpu.SemaphoreType.DMA,
            pltpu.SemaphoreType.DMA,
            pltpu.SemaphoreType.DMA,
            pltpu.SemaphoreType.DMA,
            pltpu.SemaphoreType.DMA,
        ],
        compiler_params=pltpu.CompilerParams(needs_layout_passes=False),
    )
    def k(values_hbm, seg_hbm, pooled_out, counts_out,
          val_b0, val_b1, val_b2, idx_b0, idx_b1, idx_b2, hist_b, lin_b,
          pooled_sh, counts_sh, lsem0, lsem1, lsem2, ssem0, ssem1, ssem2):
        c = lax.axis_index("c")
        s = lax.axis_index("s")
        base = (c * _NS + s) * _CHUNK

        val_b = (val_b0, val_b1, val_b2)
        idx_b = (idx_b0, idx_b1, idx_b2)
        lsem = (lsem0, lsem1, lsem2)
        ssem = (ssem0, ssem1, ssem2)

        zeros16 = jnp.zeros((16,), jnp.float32)
        ones16 = jnp.ones((16,), jnp.float32)
        iota16 = lax.iota(jnp.int32, 16)

        def start_load(i, b):
            off = _SEG_OFF + base + i * _BLK
            pltpu.async_copy(seg_hbm.at[pl.ds(off, _BLK)], idx_b[b], lsem[b])
            off = base + i * _BLK
            pltpu.async_copy(values_hbm.at[pl.ds(off, _BLK)], val_b[b], lsem[b])

        def wait_load(b):
            pltpu.make_async_copy(seg_hbm.at[pl.ds(0, _BLK)],
                                  idx_b[b], lsem[b]).wait()
            pltpu.make_async_copy(values_hbm.at[pl.ds(0, _BLK)],
                                  val_b[b], lsem[b]).wait()

        def wait_scatter(b):
            pltpu.make_async_copy(values_hbm.at[pl.ds(0, _BLK)],
                                  val_b[b], ssem[b]).wait()

        start_load(0, 0)
        start_load(1, 1)

        def fill_zero(r, _):
            for j in range(_D // 16):
                hist_b[r, pl.ds(j * 16, 16)] = zeros16
            return _

        lax.fori_loop(0, _CROWS, fill_zero, None)

        for t in range(_CROWS // 16):
            lin_b[pl.ds(t * 16, 16)] = iota16 + (16 * t)

        def zero_blocks(t, _):
            blk = t * _NS + s

            @pl.when(blk < _SEGB)
            def _do():
                pltpu.sync_copy(hist_b, pooled_sh.at[pl.ds(blk * _CROWS,
                                                           _CROWS)])

            return _

        lax.fori_loop(0, (_SEGB + _NS - 1) // _NS, zero_blocks, None)

        @pl.when(s == 0)
        def _zero_counts():
            pltpu.sync_copy(hist_b, counts_sh)

        plsc.subcore_barrier()

        def process(i, b, drain):
            wait_load(b)
            pltpu.async_copy(val_b[b], pooled_sh.at[idx_b[b]], ssem[b],
                             add=True)
            for j in range(_BLK // 16):
                idxv = idx_b[b][pl.ds(j * 16, 16)]
                row = lax.shift_right_logical(idxv, 7)
                col = lax.bitwise_and(idxv, 127)
                plsc.addupdate_scatter(hist_b, [row, col], ones16)
            if drain:
                wait_scatter((b + 2) % 3)

            @pl.when(i + 2 < _NBLK)
            def _next():
                start_load(i + 2, (b + 2) % 3)

        process(0, 0, False)
        process(1, 1, True)

        def body(g, _):
            i = g * 3 + 2
            process(i, 2, True)
            process(i + 1, 0, True)
            process(i + 2, 1, True)
            return _

        lax.fori_loop(0, (_NBLK - 2) // 3, body, None)
        wait_scatter((_NBLK - 1) % 3)
        plsc.subcore_barrier()

        pltpu.sync_copy(hist_b, counts_sh.at[lin_b], add=True)

        def writeback(t, _):
            blk = t * _NS + s

            @pl.when(blk < _SEGB)
            def _do():
                pltpu.sync_copy(pooled_sh.at[pl.ds(blk * _CROWS, _CROWS)],
                                hist_b)
                pltpu.sync_copy(hist_b, pooled_out.at[c, blk])

            return _

        lax.fori_loop(0, (_SEGB + _NS - 1) // _NS, writeback, None)
        plsc.subcore_barrier()

        @pl.when(s == 0)
        def _write_counts():
            pltpu.sync_copy(counts_sh, hist_b)
            pltpu.sync_copy(hist_b, counts_out.at[c])

    return k(values, seg)


def _tc_merge(pooled, cnt_t):
    blk = 1000

    def k(p0_ref, p1_ref, c_ref, o_ref):
        n = jnp.sum(c_ref[...], axis=1, keepdims=True) + _EPS
        o_ref[...] = (p0_ref[0] + p1_ref[0]) / n

    grid = _N_SEG // blk
    return pl.pallas_call(
        k,
        grid=(grid,),
        in_specs=[
            pl.BlockSpec((1, blk, _D), lambda i: (0, i, 0)),
            pl.BlockSpec((1, blk, _D), lambda i: (1, i, 0)),
            pl.BlockSpec((blk, _NC), lambda i: (i, 0)),
        ],
        out_specs=pl.BlockSpec((blk, _D), lambda i: (i, 0)),
        out_shape=jax.ShapeDtypeStruct((_N_SEG, _D), jnp.float32),
    )(pooled, pooled, cnt_t)


@jax.jit
def kernel(tens_values, tens_indices):
    seg = tens_indices.astype(jnp.int32).reshape(2 * _NNZ)
    pooled, counts = _sc_scatter(tens_values, seg)
    pooled = pooled.reshape(_NC, _N_SEG, _D)
    cnt_t = counts.reshape(_NC, _CROWS * _D).transpose(1, 0)
    return _tc_merge(pooled, cnt_t)
